# baseline (speedup 1.0000x reference)
import jax
import jax.numpy as jnp
from jax.experimental import pallas as pl
from jax.experimental.pallas import tpu as pltpu


_C = 1024
_K = 16


def _in_copy(z_hbm, zbuf, insem, chunk, slot):
    return pltpu.make_async_copy(
        z_hbm.at[pl.ds(chunk * _C, _C), :], zbuf.at[slot], insem.at[slot]
    )


def _out_copy(obuf, o_hbm, outsem, chunk, slot):
    return pltpu.make_async_copy(
        obuf.at[slot], o_hbm.at[:, pl.ds(chunk * _C, _C)], outsem.at[slot]
    )


def _gate_body(z_hbm, w_ref, b_ref, o_hbm, zbuf, obuf, insem, outsem):
    n_chunks = z_hbm.shape[0] // _C
    w = w_ref[...]
    b = b_ref[...]

    for s in range(_K):
        _in_copy(z_hbm, zbuf, insem, s, s).start()

    def step(i, carry):
        slot = jax.lax.rem(i, _K)
        _in_copy(z_hbm, zbuf, insem, i, slot).wait()

        @pl.when(i >= _K)
        def _():
            _out_copy(obuf, o_hbm, outsem, i - _K, slot).wait()

        z = zbuf[slot]
        logits = jax.lax.dot_general(
            z, w, (((1,), (0,)), ((), ())), preferred_element_type=jnp.float32
        )
        lt = jnp.transpose(logits) + b
        m = jnp.max(lt, axis=0, keepdims=True)
        e = jnp.exp(lt - m)
        obuf[slot] = e / jnp.sum(e, axis=0, keepdims=True)
        _out_copy(obuf, o_hbm, outsem, i, slot).start()

        @pl.when(i + _K < n_chunks)
        def _():
            _in_copy(z_hbm, zbuf, insem, i + _K, slot).start()

        return carry

    jax.lax.fori_loop(0, n_chunks, step, 0)

    for s in range(_K):
        chunk = n_chunks - _K + s
        _out_copy(obuf, o_hbm, outsem, chunk, chunk % _K).wait()


@jax.jit
def kernel(z, W, b):
    n_tokens, d_model = z.shape
    n_exp = W.shape[1]
    out_t = pl.pallas_call(
        _gate_body,
        in_specs=[
            pl.BlockSpec(memory_space=pl.ANY),
            pl.BlockSpec(memory_space=pltpu.VMEM),
            pl.BlockSpec(memory_space=pltpu.VMEM),
        ],
        out_specs=pl.BlockSpec(memory_space=pl.ANY),
        out_shape=jax.ShapeDtypeStruct((n_exp, n_tokens), jnp.float32),
        scratch_shapes=[
            pltpu.VMEM((_K, _C, d_model), jnp.float32),
            pltpu.VMEM((_K, n_exp, _C), jnp.float32),
            pltpu.SemaphoreType.DMA((_K,)),
            pltpu.SemaphoreType.DMA((_K,)),
        ],
    )(z, W, b.reshape(n_exp, 1))
    return out_t


# speedup vs baseline: 1.0254x; 1.0254x over previous
import jax
import jax.numpy as jnp
from jax.experimental import pallas as pl
from jax.experimental.pallas import tpu as pltpu


_C = 2048
_K = 8


def _in_copy(z_hbm, zbuf, insem, chunk, slot):
    return pltpu.make_async_copy(
        z_hbm.at[pl.ds(chunk * _C, _C), :], zbuf.at[slot], insem.at[slot]
    )


def _out_copy(obuf, o_hbm, outsem, chunk, slot):
    return pltpu.make_async_copy(
        obuf.at[slot], o_hbm.at[:, pl.ds(chunk * _C, _C)], outsem.at[slot]
    )


def _gate_body(z_hbm, w_ref, b_ref, o_hbm, zbuf, obuf, insem, outsem):
    n_chunks = z_hbm.shape[0] // _C
    w = w_ref[...]
    b = b_ref[...]

    for s in range(_K):
        _in_copy(z_hbm, zbuf, insem, s, s).start()

    def step(i, carry):
        slot = jax.lax.rem(i, _K)
        _in_copy(z_hbm, zbuf, insem, i, slot).wait()

        @pl.when(i >= _K)
        def _():
            _out_copy(obuf, o_hbm, outsem, i - _K, slot).wait()

        z = zbuf[slot]
        logits = jax.lax.dot_general(
            z, w, (((1,), (0,)), ((), ())), preferred_element_type=jnp.float32
        )
        lt = jnp.transpose(logits) + b
        m = jnp.max(lt, axis=0, keepdims=True)
        e = jnp.exp(lt - m)
        obuf[slot] = e / jnp.sum(e, axis=0, keepdims=True)
        _out_copy(obuf, o_hbm, outsem, i, slot).start()

        @pl.when(i + _K < n_chunks)
        def _():
            _in_copy(z_hbm, zbuf, insem, i + _K, slot).start()

        return carry

    jax.lax.fori_loop(0, n_chunks, step, 0)

    for s in range(_K):
        chunk = n_chunks - _K + s
        _out_copy(obuf, o_hbm, outsem, chunk, chunk % _K).wait()


@jax.jit
def kernel(z, W, b):
    n_tokens, d_model = z.shape
    n_exp = W.shape[1]
    out_t = pl.pallas_call(
        _gate_body,
        in_specs=[
            pl.BlockSpec(memory_space=pl.ANY),
            pl.BlockSpec(memory_space=pltpu.VMEM),
            pl.BlockSpec(memory_space=pltpu.VMEM),
        ],
        out_specs=pl.BlockSpec(memory_space=pl.ANY),
        out_shape=jax.ShapeDtypeStruct((n_exp, n_tokens), jnp.float32),
        scratch_shapes=[
            pltpu.VMEM((_K, _C, d_model), jnp.float32),
            pltpu.VMEM((_K, n_exp, _C), jnp.float32),
            pltpu.SemaphoreType.DMA((_K,)),
            pltpu.SemaphoreType.DMA((_K,)),
        ],
    )(z, W, b.reshape(n_exp, 1))
    return out_t.T
